# Initial kernel scaffold; baseline (speedup 1.0000x reference)
import jax
import jax.numpy as jnp
from jax.experimental import pallas as pl


def kernel(inputs_data, inputs_hyper, edge_index, hyper_edge_index, W1_gcn, W2_gcn, W3_gcn, att, W1_hyp, W_hgcn, alpha, W_dec):
    raise NotImplementedError("write your pallas kernel here")


# speedup vs baseline: 6.6408x; 6.6408x over previous
import functools

import jax
import jax.numpy as jnp
from jax import lax
from jax.experimental import pallas as pl
from jax.experimental.pallas import tpu as pltpu
from jax.experimental.pallas import tpu_sc as plsc

N = 10000
E = 320000
D1 = 128
EMB = 64
NUM_R = 5000

NC = 2
NS = 16
CHUNK = 125
NROW = E // CHUNK
ROWS_PER_SUB = NROW // NS
ROWS_PER_WORKER = NROW // (NC * NS)
SEG = 624
SEG_LAST = N - (NS - 1) * SEG

@functools.cache
def _mesh():
  return plsc.VectorSubcoreMesh(core_axis_name="c", subcore_axis_name="s",
                                num_cores=NC, num_subcores=NS)


def _spmm_phase(table_hbm, src_hbm, dst_hbm, zeros_hbm, out_hbm,
                src_v, dst_v, rows_v, acc, sem, sid, nrows, row_base):
  @pl.when(sid < NS - 1)
  def _():
    pltpu.sync_copy(zeros_hbm.at[pl.ds(sid * SEG, SEG)],
                    acc.at[pl.ds(sid * SEG, SEG)])

  @pl.when(sid == NS - 1)
  def _():
    pltpu.sync_copy(zeros_hbm.at[pl.ds((NS - 1) * SEG, SEG_LAST)],
                    acc.at[pl.ds((NS - 1) * SEG, SEG_LAST)])

  pltpu.sync_copy(src_hbm.at[pl.ds(row_base, nrows)], src_v.at[pl.ds(0, nrows)])
  pltpu.sync_copy(dst_hbm.at[pl.ds(row_base, nrows)], dst_v.at[pl.ds(0, nrows)])
  plsc.subcore_barrier()

  def body(j, carry):
    pltpu.async_copy(table_hbm.at[src_v.at[j]], rows_v, sem).wait()
    pltpu.sync_copy(rows_v, acc.at[dst_v.at[j]], add=True)
    return carry

  lax.fori_loop(0, nrows, body, 0)
  plsc.subcore_barrier()

  @pl.when(sid < NS - 1)
  def _():
    pltpu.sync_copy(acc.at[pl.ds(sid * SEG, SEG)],
                    out_hbm.at[pl.ds(sid * SEG, SEG)])

  @pl.when(sid == NS - 1)
  def _():
    pltpu.sync_copy(acc.at[pl.ds((NS - 1) * SEG, SEG_LAST)],
                    out_hbm.at[pl.ds((NS - 1) * SEG, SEG_LAST)])


def _sc_dual_kernel(h1_hbm, x_hbm, srcg_hbm, dstg_hbm, srch_hbm, dsth_hbm,
                    zeros_hbm, s1_hbm, sh_hbm, src_v, dst_v, rows_v, acc, sem):
  cid = lax.axis_index("c")
  sid = lax.axis_index("s")
  base = sid * ROWS_PER_SUB

  @pl.when(cid == 0)
  def _():
    _spmm_phase(h1_hbm, srcg_hbm, dstg_hbm, zeros_hbm, s1_hbm,
                src_v, dst_v, rows_v, acc, sem, sid, ROWS_PER_SUB, base)

  @pl.when(cid == 1)
  def _():
    _spmm_phase(x_hbm, srch_hbm, dsth_hbm, zeros_hbm, sh_hbm,
                src_v, dst_v, rows_v, acc, sem, sid, ROWS_PER_SUB, base)


def _sc_single_kernel(h2_hbm, srcg_hbm, dstg_hbm, zeros_hbm, s2a_hbm, s2b_hbm,
                      src_v, dst_v, rows_v, acc, sem):
  cid = lax.axis_index("c")
  sid = lax.axis_index("s")
  wid = sid * NC + cid
  base = wid * ROWS_PER_WORKER

  @pl.when(cid == 0)
  def _():
    _spmm_phase(h2_hbm, srcg_hbm, dstg_hbm, zeros_hbm, s2a_hbm,
                src_v, dst_v, rows_v, acc, sem, sid, ROWS_PER_WORKER, base)

  @pl.when(cid == 1)
  def _():
    _spmm_phase(h2_hbm, srcg_hbm, dstg_hbm, zeros_hbm, s2b_hbm,
                src_v, dst_v, rows_v, acc, sem, sid, ROWS_PER_WORKER, base)


@functools.cache
def _sc_dual():
  return pl.kernel(
      _sc_dual_kernel,
      out_type=(jax.ShapeDtypeStruct((N, EMB), jnp.float32),
                jax.ShapeDtypeStruct((N, EMB), jnp.float32)),
      mesh=_mesh(),
      compiler_params=pltpu.CompilerParams(use_tc_tiling_on_sc=False),
      scratch_types=[
          pltpu.VMEM((ROWS_PER_SUB, CHUNK), jnp.int32),
          pltpu.VMEM((ROWS_PER_SUB, CHUNK), jnp.int32),
          pltpu.VMEM((CHUNK, EMB), jnp.float32),
          pltpu.VMEM_SHARED((N, EMB), jnp.float32),
          pltpu.SemaphoreType.DMA,
      ],
  )


@functools.cache
def _sc_single():
  return pl.kernel(
      _sc_single_kernel,
      out_type=(jax.ShapeDtypeStruct((N, EMB), jnp.float32),
                jax.ShapeDtypeStruct((N, EMB), jnp.float32)),
      mesh=_mesh(),
      compiler_params=pltpu.CompilerParams(use_tc_tiling_on_sc=False),
      scratch_types=[
          pltpu.VMEM((ROWS_PER_SUB, CHUNK), jnp.int32),
          pltpu.VMEM((ROWS_PER_SUB, CHUNK), jnp.int32),
          pltpu.VMEM((CHUNK, EMB), jnp.float32),
          pltpu.VMEM_SHARED((N, EMB), jnp.float32),
          pltpu.SemaphoreType.DMA,
      ],
  )


def _elu(t):
  return jnp.where(t > 0, t, jnp.exp(t) - 1.0)


def _tc_a_body(x_ref, w1g_ref, xh_ref, w1h_ref, h1_ref, xo_ref):
  h1_ref[...] = _elu(jnp.dot(x_ref[...], w1g_ref[...],
                             preferred_element_type=jnp.float32,
                             precision=lax.Precision.HIGHEST))
  xo_ref[...] = jnp.maximum(jnp.dot(xh_ref[...], w1h_ref[...],
                                    preferred_element_type=jnp.float32,
                                    precision=lax.Precision.HIGHEST), 0.0)


_tc_a = pl.pallas_call(
    _tc_a_body,
    out_shape=(jax.ShapeDtypeStruct((N, EMB), jnp.float32),
               jax.ShapeDtypeStruct((N, EMB), jnp.float32)),
)


def _tc_b_body(s1_ref, w2_ref, sh_ref, whg_ref, h2_ref, eh_ref):
  h2_ref[...] = _elu(jnp.dot(s1_ref[...], w2_ref[...],
                             preferred_element_type=jnp.float32,
                             precision=lax.Precision.HIGHEST))
  eh_ref[...] = jnp.maximum(jnp.dot(sh_ref[...], whg_ref[...],
                                    preferred_element_type=jnp.float32,
                                    precision=lax.Precision.HIGHEST), 0.0)


_tc_b = pl.pallas_call(
    _tc_b_body,
    out_shape=(jax.ShapeDtypeStruct((N, EMB), jnp.float32),
               jax.ShapeDtypeStruct((N, EMB), jnp.float32)),
)


def _tc_c_body(s2a_ref, s2b_ref, w3_ref, h1_ref, h2_ref, eh_ref,
               att_ref, alpha_ref, wdec_ref, rw_ref, dm_ref):
  h3 = _elu(jnp.dot(s2a_ref[...] + s2b_ref[...], w3_ref[...],
                    preferred_element_type=jnp.float32,
                             precision=lax.Precision.HIGHEST))
  e0 = jnp.exp(alpha_ref[0])
  e1 = jnp.exp(alpha_ref[1])
  a0 = e0 / (e0 + e1)
  a1 = e1 / (e0 + e1)
  fused = (a0 * (att_ref[0] * h1_ref[...] + att_ref[1] * h2_ref[...]
                 + att_ref[2] * h3)
           + a1 * eh_ref[...])
  rw_ref[...] = jnp.dot(fused[:NUM_R], wdec_ref[...],
                        preferred_element_type=jnp.float32,
                      precision=lax.Precision.HIGHEST)
  dm_ref[...] = fused[NUM_R:]


_tc_c = pl.pallas_call(
    _tc_c_body,
    in_specs=[pl.BlockSpec() for _ in range(6)] + [
        pl.BlockSpec(memory_space=pltpu.SMEM),
        pl.BlockSpec(memory_space=pltpu.SMEM),
        pl.BlockSpec(),
    ],
    out_shape=(jax.ShapeDtypeStruct((NUM_R, EMB), jnp.float32),
               jax.ShapeDtypeStruct((N - NUM_R, EMB), jnp.float32)),
)

_BM = 1000


def _tc_d_body(rw_ref, dm_ref, out_ref):
  t = lax.dot_general(rw_ref[...], dm_ref[...],
                      dimension_numbers=(((1,), (1,)), ((), ())),
                      preferred_element_type=jnp.float32,
                      precision=lax.Precision.HIGHEST)
  out_ref[...] = 1.0 / (1.0 + jnp.exp(-t))


_tc_d = pl.pallas_call(
    _tc_d_body,
    grid=(NUM_R // _BM,),
    in_specs=[
        pl.BlockSpec((_BM, EMB), lambda i: (i, 0)),
        pl.BlockSpec((NUM_R, EMB), lambda i: (0, 0)),
    ],
    out_specs=pl.BlockSpec((_BM, NUM_R), lambda i: (i, 0)),
    out_shape=jax.ShapeDtypeStruct((NUM_R, NUM_R), jnp.float32),
)


def kernel(inputs_data, inputs_hyper, edge_index, hyper_edge_index,
           W1_gcn, W2_gcn, W3_gcn, att, W1_hyp, W_hgcn, alpha, W_dec):
  srcg = edge_index[0].reshape(NROW, CHUNK)
  dstg = edge_index[1].reshape(NROW, CHUNK)
  srch = hyper_edge_index[0].reshape(NROW, CHUNK)
  dsth = hyper_edge_index[1].reshape(NROW, CHUNK)
  zeros = jnp.zeros((N, EMB), jnp.float32)

  h1, x = _tc_a(inputs_data, W1_gcn, inputs_hyper, W1_hyp)
  s1, sh = _sc_dual()(h1, x, srcg, dstg, srch, dsth, zeros)
  h2, emb_hyper = _tc_b(s1, W2_gcn, sh, W_hgcn)
  s2a, s2b = _sc_single()(h2, srcg, dstg, zeros)
  rw, dm = _tc_c(s2a, s2b, W3_gcn, h1, h2, emb_hyper, att, alpha, W_dec)
  return _tc_d(rw, dm)


# speedup vs baseline: 8.9487x; 1.3475x over previous
import functools

import jax
import jax.numpy as jnp
from jax import lax
from jax.experimental import pallas as pl
from jax.experimental.pallas import tpu as pltpu
from jax.experimental.pallas import tpu_sc as plsc

N = 10000
E = 320000
D1 = 128
EMB = 64
NUM_R = 5000

NC = 2
NS = 16
CHUNK = 125
NROW = E // CHUNK
ROWS_PER_SUB = NROW // NS
ROWS_PER_WORKER = NROW // (NC * NS)
SEG = 624
SEG_LAST = N - (NS - 1) * SEG

@functools.cache
def _mesh():
  return plsc.VectorSubcoreMesh(core_axis_name="c", subcore_axis_name="s",
                                num_cores=NC, num_subcores=NS)


def _spmm_phase(table_hbm, src_hbm, dst_hbm, zeros_hbm, out_hbm,
                src_v, dst_v, rows0, rows1, acc, semg0, semg1,
                sid, nrows, row_base):
  @pl.when(sid < NS - 1)
  def _():
    pltpu.sync_copy(zeros_hbm.at[pl.ds(sid * SEG, SEG)],
                    acc.at[pl.ds(sid * SEG, SEG)])

  @pl.when(sid == NS - 1)
  def _():
    pltpu.sync_copy(zeros_hbm.at[pl.ds((NS - 1) * SEG, SEG_LAST)],
                    acc.at[pl.ds((NS - 1) * SEG, SEG_LAST)])

  pltpu.sync_copy(src_hbm.at[pl.ds(row_base, nrows)], src_v.at[pl.ds(0, nrows)])
  pltpu.sync_copy(dst_hbm.at[pl.ds(row_base, nrows)], dst_v.at[pl.ds(0, nrows)])
  plsc.subcore_barrier()

  pltpu.async_copy(table_hbm.at[src_v.at[0]], rows0, semg0)

  def body(k, carry):
    j0 = 2 * k
    cp1 = pltpu.async_copy(table_hbm.at[src_v.at[j0 + 1]], rows1, semg1)
    pltpu.make_async_copy(table_hbm.at[src_v.at[j0]], rows0, semg0).wait()
    pltpu.sync_copy(rows0, acc.at[dst_v.at[j0]], add=True)

    @pl.when(j0 + 2 < nrows)
    def _():
      pltpu.async_copy(table_hbm.at[src_v.at[j0 + 2]], rows0, semg0)

    cp1.wait()
    pltpu.sync_copy(rows1, acc.at[dst_v.at[j0 + 1]], add=True)
    return carry

  lax.fori_loop(0, nrows // 2, body, 0)
  plsc.subcore_barrier()

  @pl.when(sid < NS - 1)
  def _():
    pltpu.sync_copy(acc.at[pl.ds(sid * SEG, SEG)],
                    out_hbm.at[pl.ds(sid * SEG, SEG)])

  @pl.when(sid == NS - 1)
  def _():
    pltpu.sync_copy(acc.at[pl.ds((NS - 1) * SEG, SEG_LAST)],
                    out_hbm.at[pl.ds((NS - 1) * SEG, SEG_LAST)])


def _sc_dual_kernel(h1_hbm, x_hbm, srcg_hbm, dstg_hbm, srch_hbm, dsth_hbm,
                    zeros_hbm, s1_hbm, sh_hbm, src_v, dst_v, rows0, rows1,
                    acc, semg0, semg1):
  cid = lax.axis_index("c")
  sid = lax.axis_index("s")
  base = sid * ROWS_PER_SUB

  @pl.when(cid == 0)
  def _():
    _spmm_phase(h1_hbm, srcg_hbm, dstg_hbm, zeros_hbm, s1_hbm,
                src_v, dst_v, rows0, rows1, acc, semg0, semg1,
                sid, ROWS_PER_SUB, base)

  @pl.when(cid == 1)
  def _():
    _spmm_phase(x_hbm, srch_hbm, dsth_hbm, zeros_hbm, sh_hbm,
                src_v, dst_v, rows0, rows1, acc, semg0, semg1,
                sid, ROWS_PER_SUB, base)


def _sc_single_kernel(h2_hbm, srcg_hbm, dstg_hbm, zeros_hbm, s2a_hbm, s2b_hbm,
                      src_v, dst_v, rows0, rows1, acc, semg0, semg1):
  cid = lax.axis_index("c")
  sid = lax.axis_index("s")
  wid = sid * NC + cid
  base = wid * ROWS_PER_WORKER

  @pl.when(cid == 0)
  def _():
    _spmm_phase(h2_hbm, srcg_hbm, dstg_hbm, zeros_hbm, s2a_hbm,
                src_v, dst_v, rows0, rows1, acc, semg0, semg1,
                sid, ROWS_PER_WORKER, base)

  @pl.when(cid == 1)
  def _():
    _spmm_phase(h2_hbm, srcg_hbm, dstg_hbm, zeros_hbm, s2b_hbm,
                src_v, dst_v, rows0, rows1, acc, semg0, semg1,
                sid, ROWS_PER_WORKER, base)


@functools.cache
def _sc_dual():
  return pl.kernel(
      _sc_dual_kernel,
      out_type=(jax.ShapeDtypeStruct((N, EMB), jnp.float32),
                jax.ShapeDtypeStruct((N, EMB), jnp.float32)),
      mesh=_mesh(),
      compiler_params=pltpu.CompilerParams(use_tc_tiling_on_sc=False),
      scratch_types=[
          pltpu.VMEM((ROWS_PER_SUB, CHUNK), jnp.int32),
          pltpu.VMEM((ROWS_PER_SUB, CHUNK), jnp.int32),
          pltpu.VMEM((CHUNK, EMB), jnp.float32),
          pltpu.VMEM((CHUNK, EMB), jnp.float32),
          pltpu.VMEM_SHARED((N, EMB), jnp.float32),
          pltpu.SemaphoreType.DMA,
          pltpu.SemaphoreType.DMA,
      ],
  )


@functools.cache
def _sc_single():
  return pl.kernel(
      _sc_single_kernel,
      out_type=(jax.ShapeDtypeStruct((N, EMB), jnp.float32),
                jax.ShapeDtypeStruct((N, EMB), jnp.float32)),
      mesh=_mesh(),
      compiler_params=pltpu.CompilerParams(use_tc_tiling_on_sc=False),
      scratch_types=[
          pltpu.VMEM((ROWS_PER_SUB, CHUNK), jnp.int32),
          pltpu.VMEM((ROWS_PER_SUB, CHUNK), jnp.int32),
          pltpu.VMEM((CHUNK, EMB), jnp.float32),
          pltpu.VMEM((CHUNK, EMB), jnp.float32),
          pltpu.VMEM_SHARED((N, EMB), jnp.float32),
          pltpu.SemaphoreType.DMA,
          pltpu.SemaphoreType.DMA,
      ],
  )


def _elu(t):
  return jnp.where(t > 0, t, jnp.exp(t) - 1.0)


def _tc_a_body(x_ref, w1g_ref, xh_ref, w1h_ref, h1_ref, xo_ref):
  h1_ref[...] = _elu(jnp.dot(x_ref[...], w1g_ref[...],
                             preferred_element_type=jnp.float32,
                             precision=lax.Precision.HIGHEST))
  xo_ref[...] = jnp.maximum(jnp.dot(xh_ref[...], w1h_ref[...],
                                    preferred_element_type=jnp.float32,
                                    precision=lax.Precision.HIGHEST), 0.0)


_tc_a = pl.pallas_call(
    _tc_a_body,
    out_shape=(jax.ShapeDtypeStruct((N, EMB), jnp.float32),
               jax.ShapeDtypeStruct((N, EMB), jnp.float32)),
)


def _tc_b_body(s1_ref, w2_ref, sh_ref, whg_ref, h2_ref, eh_ref):
  h2_ref[...] = _elu(jnp.dot(s1_ref[...], w2_ref[...],
                             preferred_element_type=jnp.float32,
                             precision=lax.Precision.HIGHEST))
  eh_ref[...] = jnp.maximum(jnp.dot(sh_ref[...], whg_ref[...],
                                    preferred_element_type=jnp.float32,
                                    precision=lax.Precision.HIGHEST), 0.0)


_tc_b = pl.pallas_call(
    _tc_b_body,
    out_shape=(jax.ShapeDtypeStruct((N, EMB), jnp.float32),
               jax.ShapeDtypeStruct((N, EMB), jnp.float32)),
)


def _tc_cd_body(s2a_ref, s2b_ref, w3_ref, h1_ref, h2_ref, eh_ref,
                att_ref, alpha_ref, wdec_ref, out_ref, rw_s, dm_s):
  i = pl.program_id(0)

  @pl.when(i == 0)
  def _():
    h3 = _elu(jnp.dot(s2a_ref[...] + s2b_ref[...], w3_ref[...],
                      preferred_element_type=jnp.float32,
                      precision=lax.Precision.HIGHEST))
    e0 = jnp.exp(alpha_ref[0])
    e1 = jnp.exp(alpha_ref[1])
    a0 = e0 / (e0 + e1)
    a1 = e1 / (e0 + e1)
    fused = (a0 * (att_ref[0] * h1_ref[...] + att_ref[1] * h2_ref[...]
                   + att_ref[2] * h3)
             + a1 * eh_ref[...])
    rw_s[...] = jnp.dot(fused[:NUM_R], wdec_ref[...],
                        preferred_element_type=jnp.float32,
                        precision=lax.Precision.HIGHEST)
    dm_s[...] = fused[NUM_R:]

  t = lax.dot_general(rw_s[pl.ds(i * _BM, _BM), :], dm_s[...],
                      dimension_numbers=(((1,), (1,)), ((), ())),
                      preferred_element_type=jnp.float32,
                      precision=lax.Precision.HIGHEST)
  out_ref[...] = 1.0 / (1.0 + jnp.exp(-t))


_BM = 200

_tc_cd = pl.pallas_call(
    _tc_cd_body,
    grid=(NUM_R // _BM,),
    in_specs=[pl.BlockSpec((N, EMB), lambda i: (0, 0)) for _ in range(2)] + [
        pl.BlockSpec((EMB, EMB), lambda i: (0, 0)),
    ] + [pl.BlockSpec((N, EMB), lambda i: (0, 0)) for _ in range(3)] + [
        pl.BlockSpec(memory_space=pltpu.SMEM),
        pl.BlockSpec(memory_space=pltpu.SMEM),
        pl.BlockSpec((EMB, EMB), lambda i: (0, 0)),
    ],
    out_specs=pl.BlockSpec((_BM, NUM_R), lambda i: (i, 0)),
    out_shape=jax.ShapeDtypeStruct((NUM_R, NUM_R), jnp.float32),
    scratch_shapes=[
        pltpu.VMEM((NUM_R, EMB), jnp.float32),
        pltpu.VMEM((NUM_R, EMB), jnp.float32),
    ],
)


def kernel(inputs_data, inputs_hyper, edge_index, hyper_edge_index,
           W1_gcn, W2_gcn, W3_gcn, att, W1_hyp, W_hgcn, alpha, W_dec):
  srcg = edge_index[0].reshape(NROW, CHUNK)
  dstg = edge_index[1].reshape(NROW, CHUNK)
  srch = hyper_edge_index[0].reshape(NROW, CHUNK)
  dsth = hyper_edge_index[1].reshape(NROW, CHUNK)
  zeros = jnp.zeros((N, EMB), jnp.float32)

  h1, x = _tc_a(inputs_data, W1_gcn, inputs_hyper, W1_hyp)
  s1, sh = _sc_dual()(h1, x, srcg, dstg, srch, dsth, zeros)
  h2, emb_hyper = _tc_b(s1, W2_gcn, sh, W_hgcn)
  s2a, s2b = _sc_single()(h2, srcg, dstg, zeros)
  return _tc_cd(s2a, s2b, W3_gcn, h1, h2, emb_hyper, att, alpha, W_dec)


# speedup vs baseline: 9.1909x; 1.0271x over previous
import functools

import jax
import jax.numpy as jnp
from jax import lax
from jax.experimental import pallas as pl
from jax.experimental.pallas import tpu as pltpu
from jax.experimental.pallas import tpu_sc as plsc

N = 10000
E = 320000
D1 = 128
EMB = 64
NUM_R = 5000

NC = 2
NS = 16
CHUNK = 125
NROW = E // CHUNK
ROWS_PER_SUB = NROW // NS
ROWS_PER_WORKER = NROW // (NC * NS)
SEG = 624
SEG_LAST = N - (NS - 1) * SEG

@functools.cache
def _mesh():
  return plsc.VectorSubcoreMesh(core_axis_name="c", subcore_axis_name="s",
                                num_cores=NC, num_subcores=NS)


def _spmm_phase(table_hbm, src_hbm, dst_hbm, zeros_hbm, out_hbm,
                src_v, dst_v, rows0, rows1, acc, semg0, semg1,
                sid, nrows, row_base):
  @pl.when(sid < NS - 1)
  def _():
    pltpu.sync_copy(zeros_hbm.at[pl.ds(sid * SEG, SEG)],
                    acc.at[pl.ds(sid * SEG, SEG)])

  @pl.when(sid == NS - 1)
  def _():
    pltpu.sync_copy(zeros_hbm.at[pl.ds((NS - 1) * SEG, SEG_LAST)],
                    acc.at[pl.ds((NS - 1) * SEG, SEG_LAST)])

  pltpu.sync_copy(src_hbm.at[pl.ds(row_base, nrows)], src_v.at[pl.ds(0, nrows)])
  pltpu.sync_copy(dst_hbm.at[pl.ds(row_base, nrows)], dst_v.at[pl.ds(0, nrows)])
  plsc.subcore_barrier()

  pltpu.async_copy(table_hbm.at[src_v.at[0]], rows0, semg0)

  def body(k, carry):
    j0 = 2 * k
    cp1 = pltpu.async_copy(table_hbm.at[src_v.at[j0 + 1]], rows1, semg1)
    pltpu.make_async_copy(table_hbm.at[src_v.at[j0]], rows0, semg0).wait()
    pltpu.sync_copy(rows0, acc.at[dst_v.at[j0]], add=True)

    @pl.when(j0 + 2 < nrows)
    def _():
      pltpu.async_copy(table_hbm.at[src_v.at[j0 + 2]], rows0, semg0)

    cp1.wait()
    pltpu.sync_copy(rows1, acc.at[dst_v.at[j0 + 1]], add=True)
    return carry

  lax.fori_loop(0, nrows // 2, body, 0)
  plsc.subcore_barrier()

  @pl.when(sid < NS - 1)
  def _():
    pltpu.sync_copy(acc.at[pl.ds(sid * SEG, SEG)],
                    out_hbm.at[pl.ds(sid * SEG, SEG)])

  @pl.when(sid == NS - 1)
  def _():
    pltpu.sync_copy(acc.at[pl.ds((NS - 1) * SEG, SEG_LAST)],
                    out_hbm.at[pl.ds((NS - 1) * SEG, SEG_LAST)])


def _sc_dual_kernel(h1_hbm, x_hbm, srcg_hbm, dstg_hbm, srch_hbm, dsth_hbm,
                    zeros_hbm, s1_hbm, sh_hbm, src_v, dst_v, rows0, rows1,
                    acc, semg0, semg1):
  cid = lax.axis_index("c")
  sid = lax.axis_index("s")
  base = sid * ROWS_PER_SUB

  @pl.when(cid == 0)
  def _():
    _spmm_phase(h1_hbm, srcg_hbm, dstg_hbm, zeros_hbm, s1_hbm,
                src_v, dst_v, rows0, rows1, acc, semg0, semg1,
                sid, ROWS_PER_SUB, base)

  @pl.when(cid == 1)
  def _():
    _spmm_phase(x_hbm, srch_hbm, dsth_hbm, zeros_hbm, sh_hbm,
                src_v, dst_v, rows0, rows1, acc, semg0, semg1,
                sid, ROWS_PER_SUB, base)


def _sc_single_kernel(h2_hbm, srcg_hbm, dstg_hbm, zeros_hbm, s2a_hbm, s2b_hbm,
                      src_v, dst_v, rows0, rows1, acc, semg0, semg1):
  cid = lax.axis_index("c")
  sid = lax.axis_index("s")
  wid = sid * NC + cid
  base = wid * ROWS_PER_WORKER

  @pl.when(cid == 0)
  def _():
    _spmm_phase(h2_hbm, srcg_hbm, dstg_hbm, zeros_hbm, s2a_hbm,
                src_v, dst_v, rows0, rows1, acc, semg0, semg1,
                sid, ROWS_PER_WORKER, base)

  @pl.when(cid == 1)
  def _():
    _spmm_phase(h2_hbm, srcg_hbm, dstg_hbm, zeros_hbm, s2b_hbm,
                src_v, dst_v, rows0, rows1, acc, semg0, semg1,
                sid, ROWS_PER_WORKER, base)


@functools.cache
def _sc_dual():
  return pl.kernel(
      _sc_dual_kernel,
      out_type=(jax.ShapeDtypeStruct((N, EMB), jnp.float32),
                jax.ShapeDtypeStruct((N, EMB), jnp.float32)),
      mesh=_mesh(),
      compiler_params=pltpu.CompilerParams(use_tc_tiling_on_sc=False),
      scratch_types=[
          pltpu.VMEM((ROWS_PER_SUB, CHUNK), jnp.int32),
          pltpu.VMEM((ROWS_PER_SUB, CHUNK), jnp.int32),
          pltpu.VMEM((CHUNK, EMB), jnp.float32),
          pltpu.VMEM((CHUNK, EMB), jnp.float32),
          pltpu.VMEM_SHARED((N, EMB), jnp.float32),
          pltpu.SemaphoreType.DMA,
          pltpu.SemaphoreType.DMA,
      ],
  )


@functools.cache
def _sc_single():
  return pl.kernel(
      _sc_single_kernel,
      out_type=(jax.ShapeDtypeStruct((N, EMB), jnp.float32),
                jax.ShapeDtypeStruct((N, EMB), jnp.float32)),
      mesh=_mesh(),
      compiler_params=pltpu.CompilerParams(use_tc_tiling_on_sc=False),
      scratch_types=[
          pltpu.VMEM((ROWS_PER_SUB, CHUNK), jnp.int32),
          pltpu.VMEM((ROWS_PER_SUB, CHUNK), jnp.int32),
          pltpu.VMEM((CHUNK, EMB), jnp.float32),
          pltpu.VMEM((CHUNK, EMB), jnp.float32),
          pltpu.VMEM_SHARED((N, EMB), jnp.float32),
          pltpu.SemaphoreType.DMA,
          pltpu.SemaphoreType.DMA,
      ],
  )


def _elu(t):
  return jnp.where(t > 0, t, jnp.exp(t) - 1.0)


def _tc_a_body(x_ref, w1g_ref, xh_ref, w1h_ref, h1_ref, xo_ref):
  h1_ref[...] = _elu(jnp.dot(x_ref[...], w1g_ref[...],
                             preferred_element_type=jnp.float32,
                             precision=lax.Precision.HIGHEST))
  xo_ref[...] = jnp.maximum(jnp.dot(xh_ref[...], w1h_ref[...],
                                    preferred_element_type=jnp.float32,
                                    precision=lax.Precision.HIGHEST), 0.0)


_RB = 2000

_tc_a = pl.pallas_call(
    _tc_a_body,
    grid=(N // _RB,),
    in_specs=[
        pl.BlockSpec((_RB, D1), lambda i: (i, 0)),
        pl.BlockSpec((D1, EMB), lambda i: (0, 0)),
        pl.BlockSpec((_RB, D1), lambda i: (i, 0)),
        pl.BlockSpec((D1, EMB), lambda i: (0, 0)),
    ],
    out_specs=(pl.BlockSpec((_RB, EMB), lambda i: (i, 0)),
               pl.BlockSpec((_RB, EMB), lambda i: (i, 0))),
    out_shape=(jax.ShapeDtypeStruct((N, EMB), jnp.float32),
               jax.ShapeDtypeStruct((N, EMB), jnp.float32)),
)


def _tc_b_body(s1_ref, w2_ref, sh_ref, whg_ref, h2_ref, eh_ref):
  h2_ref[...] = _elu(jnp.dot(s1_ref[...], w2_ref[...],
                             preferred_element_type=jnp.float32,
                             precision=lax.Precision.HIGHEST))
  eh_ref[...] = jnp.maximum(jnp.dot(sh_ref[...], whg_ref[...],
                                    preferred_element_type=jnp.float32,
                                    precision=lax.Precision.HIGHEST), 0.0)


_tc_b = pl.pallas_call(
    _tc_b_body,
    grid=(N // _RB,),
    in_specs=[
        pl.BlockSpec((_RB, EMB), lambda i: (i, 0)),
        pl.BlockSpec((EMB, EMB), lambda i: (0, 0)),
        pl.BlockSpec((_RB, EMB), lambda i: (i, 0)),
        pl.BlockSpec((EMB, EMB), lambda i: (0, 0)),
    ],
    out_specs=(pl.BlockSpec((_RB, EMB), lambda i: (i, 0)),
               pl.BlockSpec((_RB, EMB), lambda i: (i, 0))),
    out_shape=(jax.ShapeDtypeStruct((N, EMB), jnp.float32),
               jax.ShapeDtypeStruct((N, EMB), jnp.float32)),
)


def _tc_cd_body(s2a_ref, s2b_ref, w3_ref, h1_ref, h2_ref, eh_ref,
                att_ref, alpha_ref, wdec_ref, out_ref, rw_s, dm_s):
  i = pl.program_id(0)

  @pl.when(i == 0)
  def _():
    h3 = _elu(jnp.dot(s2a_ref[...] + s2b_ref[...], w3_ref[...],
                      preferred_element_type=jnp.float32,
                      precision=lax.Precision.HIGHEST))
    e0 = jnp.exp(alpha_ref[0])
    e1 = jnp.exp(alpha_ref[1])
    a0 = e0 / (e0 + e1)
    a1 = e1 / (e0 + e1)
    fused = (a0 * (att_ref[0] * h1_ref[...] + att_ref[1] * h2_ref[...]
                   + att_ref[2] * h3)
             + a1 * eh_ref[...])
    rw_s[...] = jnp.dot(fused[:NUM_R], wdec_ref[...],
                        preferred_element_type=jnp.float32,
                        precision=lax.Precision.HIGHEST)
    dm_s[...] = fused[NUM_R:]

  t = lax.dot_general(rw_s[pl.ds(i * _BM, _BM), :], dm_s[...],
                      dimension_numbers=(((1,), (1,)), ((), ())),
                      preferred_element_type=jnp.float32,
                      precision=lax.Precision.HIGHEST)
  out_ref[...] = 1.0 / (1.0 + jnp.exp(-t))


_BM = 200

_tc_cd = pl.pallas_call(
    _tc_cd_body,
    grid=(NUM_R // _BM,),
    in_specs=[pl.BlockSpec((N, EMB), lambda i: (0, 0)) for _ in range(2)] + [
        pl.BlockSpec((EMB, EMB), lambda i: (0, 0)),
    ] + [pl.BlockSpec((N, EMB), lambda i: (0, 0)) for _ in range(3)] + [
        pl.BlockSpec(memory_space=pltpu.SMEM),
        pl.BlockSpec(memory_space=pltpu.SMEM),
        pl.BlockSpec((EMB, EMB), lambda i: (0, 0)),
    ],
    out_specs=pl.BlockSpec((_BM, NUM_R), lambda i: (i, 0)),
    out_shape=jax.ShapeDtypeStruct((NUM_R, NUM_R), jnp.float32),
    scratch_shapes=[
        pltpu.VMEM((NUM_R, EMB), jnp.float32),
        pltpu.VMEM((NUM_R, EMB), jnp.float32),
    ],
)


def kernel(inputs_data, inputs_hyper, edge_index, hyper_edge_index,
           W1_gcn, W2_gcn, W3_gcn, att, W1_hyp, W_hgcn, alpha, W_dec):
  srcg = edge_index[0].reshape(NROW, CHUNK)
  dstg = edge_index[1].reshape(NROW, CHUNK)
  srch = hyper_edge_index[0].reshape(NROW, CHUNK)
  dsth = hyper_edge_index[1].reshape(NROW, CHUNK)
  zeros = jnp.zeros((N, EMB), jnp.float32)

  h1, x = _tc_a(inputs_data, W1_gcn, inputs_hyper, W1_hyp)
  s1, sh = _sc_dual()(h1, x, srcg, dstg, srch, dsth, zeros)
  h2, emb_hyper = _tc_b(s1, W2_gcn, sh, W_hgcn)
  s2a, s2b = _sc_single()(h2, srcg, dstg, zeros)
  return _tc_cd(s2a, s2b, W3_gcn, h1, h2, emb_hyper, att, alpha, W_dec)


# speedup vs baseline: 10.5042x; 1.1429x over previous
import functools

import jax
import jax.numpy as jnp
from jax import lax
from jax.experimental import pallas as pl
from jax.experimental.pallas import tpu as pltpu
from jax.experimental.pallas import tpu_sc as plsc

N = 10000
E = 320000
D1 = 128
EMB = 64
NUM_R = 5000

NC = 2
NS = 16
CHUNK = 125
NROW = E // CHUNK
ROWS_PER_SUB = NROW // NS
ROWS_PER_WORKER = NROW // (NC * NS)
SEG = 624
SEG_LAST = N - (NS - 1) * SEG

@functools.cache
def _mesh():
  return plsc.VectorSubcoreMesh(core_axis_name="c", subcore_axis_name="s",
                                num_cores=NC, num_subcores=NS)


NBUF = 4


def _spmm_phase(table_hbm, edges_hbm, zeros_hbm, out_hbm,
                src_v, dst_v, rows, acc, semg, sems,
                sid, nrows, row_base):
  @pl.when(sid < NS - 1)
  def _():
    pltpu.sync_copy(zeros_hbm.at[pl.ds(sid * SEG, SEG)],
                    acc.at[pl.ds(sid * SEG, SEG)])

  @pl.when(sid == NS - 1)
  def _():
    pltpu.sync_copy(zeros_hbm.at[pl.ds((NS - 1) * SEG, SEG_LAST)],
                    acc.at[pl.ds((NS - 1) * SEG, SEG_LAST)])

  pltpu.sync_copy(edges_hbm.at[0, pl.ds(row_base, nrows)],
                  src_v.at[pl.ds(0, nrows)])
  pltpu.sync_copy(edges_hbm.at[1, pl.ds(row_base, nrows)],
                  dst_v.at[pl.ds(0, nrows)])
  plsc.subcore_barrier()

  for b in range(NBUF - 1):
    pltpu.async_copy(table_hbm.at[src_v.at[b]], rows[b], semg[b])

  def body(k, carry):
    for b in range(NBUF):
      j = NBUF * k + b
      pltpu.make_async_copy(table_hbm.at[src_v.at[j]], rows[b],
                            semg[b]).wait()
      pltpu.async_copy(rows[b], acc.at[dst_v.at[j]], sems[b], add=True)

      bp = (b + NBUF - 1) % NBUF

      @pl.when(j >= 1)
      def _():
        pltpu.make_async_copy(rows[bp], acc.at[dst_v.at[j - 1]],
                              sems[bp]).wait()

      bn = (b + NBUF - 1) % NBUF

      @pl.when(j + NBUF - 1 < nrows)
      def _():
        pltpu.async_copy(table_hbm.at[src_v.at[j + NBUF - 1]], rows[bn],
                         semg[bn])
    return carry

  lax.fori_loop(0, nrows // NBUF, body, 0)
  last = nrows - 1
  pltpu.make_async_copy(rows[(NBUF - 1) % NBUF], acc.at[dst_v.at[last]],
                        sems[(NBUF - 1) % NBUF]).wait()
  plsc.subcore_barrier()

  @pl.when(sid < NS - 1)
  def _():
    pltpu.sync_copy(acc.at[pl.ds(sid * SEG, SEG)],
                    out_hbm.at[pl.ds(sid * SEG, SEG)])

  @pl.when(sid == NS - 1)
  def _():
    pltpu.sync_copy(acc.at[pl.ds((NS - 1) * SEG, SEG_LAST)],
                    out_hbm.at[pl.ds((NS - 1) * SEG, SEG_LAST)])


def _sc_dual_kernel(h1_hbm, x_hbm, eg_hbm, eh_hbm, zeros_hbm,
                    s1_hbm, sh_hbm, src_v, dst_v,
                    r0, r1, r2, r3, acc,
                    g0, g1, g2, g3, s0, s1, s2, s3):
  cid = lax.axis_index("c")
  sid = lax.axis_index("s")
  base = sid * ROWS_PER_SUB
  rows = (r0, r1, r2, r3)
  semg = (g0, g1, g2, g3)
  sems = (s0, s1, s2, s3)

  @pl.when(cid == 0)
  def _():
    _spmm_phase(h1_hbm, eg_hbm, zeros_hbm, s1_hbm,
                src_v, dst_v, rows, acc, semg, sems,
                sid, ROWS_PER_SUB, base)

  @pl.when(cid == 1)
  def _():
    _spmm_phase(x_hbm, eh_hbm, zeros_hbm, sh_hbm,
                src_v, dst_v, rows, acc, semg, sems,
                sid, ROWS_PER_SUB, base)


def _sc_single_kernel(h2_hbm, eg_hbm, zeros_hbm, s2a_hbm, s2b_hbm,
                      src_v, dst_v,
                      r0, r1, r2, r3, acc,
                      g0, g1, g2, g3, s0, s1, s2, s3):
  cid = lax.axis_index("c")
  sid = lax.axis_index("s")
  wid = sid * NC + cid
  base = wid * ROWS_PER_WORKER
  rows = (r0, r1, r2, r3)
  semg = (g0, g1, g2, g3)
  sems = (s0, s1, s2, s3)

  @pl.when(cid == 0)
  def _():
    _spmm_phase(h2_hbm, eg_hbm, zeros_hbm, s2a_hbm,
                src_v, dst_v, rows, acc, semg, sems,
                sid, ROWS_PER_WORKER, base)

  @pl.when(cid == 1)
  def _():
    _spmm_phase(h2_hbm, eg_hbm, zeros_hbm, s2b_hbm,
                src_v, dst_v, rows, acc, semg, sems,
                sid, ROWS_PER_WORKER, base)


@functools.cache
def _sc_dual():
  return pl.kernel(
      _sc_dual_kernel,
      out_type=(jax.ShapeDtypeStruct((N, EMB), jnp.float32),
                jax.ShapeDtypeStruct((N, EMB), jnp.float32)),
      mesh=_mesh(),
      compiler_params=pltpu.CompilerParams(use_tc_tiling_on_sc=False),
      scratch_types=(
          [pltpu.VMEM((ROWS_PER_SUB, CHUNK), jnp.int32)] * 2
          + [pltpu.VMEM((CHUNK, EMB), jnp.float32)] * 4
          + [pltpu.VMEM_SHARED((N, EMB), jnp.float32)]
          + [pltpu.SemaphoreType.DMA] * 8
      ),
  )


@functools.cache
def _sc_single():
  return pl.kernel(
      _sc_single_kernel,
      out_type=(jax.ShapeDtypeStruct((N, EMB), jnp.float32),
                jax.ShapeDtypeStruct((N, EMB), jnp.float32)),
      mesh=_mesh(),
      compiler_params=pltpu.CompilerParams(use_tc_tiling_on_sc=False),
      scratch_types=(
          [pltpu.VMEM((ROWS_PER_SUB, CHUNK), jnp.int32)] * 2
          + [pltpu.VMEM((CHUNK, EMB), jnp.float32)] * 4
          + [pltpu.VMEM_SHARED((N, EMB), jnp.float32)]
          + [pltpu.SemaphoreType.DMA] * 8
      ),
  )


def _elu(t):
  return jnp.where(t > 0, t, jnp.exp(t) - 1.0)


def _tc_a_body(x_ref, w1g_ref, xh_ref, w1h_ref, h1_ref, xo_ref):
  h1_ref[...] = _elu(jnp.dot(x_ref[...], w1g_ref[...],
                             preferred_element_type=jnp.float32,
                             precision=lax.Precision.HIGHEST))
  xo_ref[...] = jnp.maximum(jnp.dot(xh_ref[...], w1h_ref[...],
                                    preferred_element_type=jnp.float32,
                                    precision=lax.Precision.HIGHEST), 0.0)


_RB = 2000

_tc_a = pl.pallas_call(
    _tc_a_body,
    grid=(N // _RB,),
    in_specs=[
        pl.BlockSpec((_RB, D1), lambda i: (i, 0)),
        pl.BlockSpec((D1, EMB), lambda i: (0, 0)),
        pl.BlockSpec((_RB, D1), lambda i: (i, 0)),
        pl.BlockSpec((D1, EMB), lambda i: (0, 0)),
    ],
    out_specs=(pl.BlockSpec((_RB, EMB), lambda i: (i, 0)),
               pl.BlockSpec((_RB, EMB), lambda i: (i, 0))),
    out_shape=(jax.ShapeDtypeStruct((N, EMB), jnp.float32),
               jax.ShapeDtypeStruct((N, EMB), jnp.float32)),
)


def _tc_b_body(s1_ref, w2_ref, sh_ref, whg_ref, h2_ref, eh_ref):
  h2_ref[...] = _elu(jnp.dot(s1_ref[...], w2_ref[...],
                             preferred_element_type=jnp.float32,
                             precision=lax.Precision.HIGHEST))
  eh_ref[...] = jnp.maximum(jnp.dot(sh_ref[...], whg_ref[...],
                                    preferred_element_type=jnp.float32,
                                    precision=lax.Precision.HIGHEST), 0.0)


_tc_b = pl.pallas_call(
    _tc_b_body,
    grid=(N // _RB,),
    in_specs=[
        pl.BlockSpec((_RB, EMB), lambda i: (i, 0)),
        pl.BlockSpec((EMB, EMB), lambda i: (0, 0)),
        pl.BlockSpec((_RB, EMB), lambda i: (i, 0)),
        pl.BlockSpec((EMB, EMB), lambda i: (0, 0)),
    ],
    out_specs=(pl.BlockSpec((_RB, EMB), lambda i: (i, 0)),
               pl.BlockSpec((_RB, EMB), lambda i: (i, 0))),
    out_shape=(jax.ShapeDtypeStruct((N, EMB), jnp.float32),
               jax.ShapeDtypeStruct((N, EMB), jnp.float32)),
)


def _tc_cd_body(s2a_ref, s2b_ref, w3_ref, h1_ref, h2_ref, eh_ref,
                att_ref, alpha_ref, wdec_ref, out_ref, rw_s, dm_s):
  i = pl.program_id(0)

  @pl.when(i == 0)
  def _():
    h3 = _elu(jnp.dot(s2a_ref[...] + s2b_ref[...], w3_ref[...],
                      preferred_element_type=jnp.float32,
                      precision=lax.Precision.HIGHEST))
    e0 = jnp.exp(alpha_ref[0])
    e1 = jnp.exp(alpha_ref[1])
    a0 = e0 / (e0 + e1)
    a1 = e1 / (e0 + e1)
    fused = (a0 * (att_ref[0] * h1_ref[...] + att_ref[1] * h2_ref[...]
                   + att_ref[2] * h3)
             + a1 * eh_ref[...])
    rw_s[...] = jnp.dot(fused[:NUM_R], wdec_ref[...],
                        preferred_element_type=jnp.float32,
                        precision=lax.Precision.HIGHEST)
    dm_s[...] = fused[NUM_R:]

  t = lax.dot_general(rw_s[pl.ds(i * _BM, _BM), :], dm_s[...],
                      dimension_numbers=(((1,), (1,)), ((), ())),
                      preferred_element_type=jnp.float32,
                      precision=lax.Precision.HIGHEST)
  out_ref[...] = 1.0 / (1.0 + jnp.exp(-t))


_BM = 200

_tc_cd = pl.pallas_call(
    _tc_cd_body,
    grid=(NUM_R // _BM,),
    in_specs=[pl.BlockSpec((N, EMB), lambda i: (0, 0)) for _ in range(2)] + [
        pl.BlockSpec((EMB, EMB), lambda i: (0, 0)),
    ] + [pl.BlockSpec((N, EMB), lambda i: (0, 0)) for _ in range(3)] + [
        pl.BlockSpec(memory_space=pltpu.SMEM),
        pl.BlockSpec(memory_space=pltpu.SMEM),
        pl.BlockSpec((EMB, EMB), lambda i: (0, 0)),
    ],
    out_specs=pl.BlockSpec((_BM, NUM_R), lambda i: (i, 0)),
    out_shape=jax.ShapeDtypeStruct((NUM_R, NUM_R), jnp.float32),
    scratch_shapes=[
        pltpu.VMEM((NUM_R, EMB), jnp.float32),
        pltpu.VMEM((NUM_R, EMB), jnp.float32),
    ],
)


def kernel(inputs_data, inputs_hyper, edge_index, hyper_edge_index,
           W1_gcn, W2_gcn, W3_gcn, att, W1_hyp, W_hgcn, alpha, W_dec):
  eg = edge_index.reshape(2, NROW, CHUNK)
  eh = hyper_edge_index.reshape(2, NROW, CHUNK)
  zeros = jnp.zeros((N, EMB), jnp.float32)

  h1, x = _tc_a(inputs_data, W1_gcn, inputs_hyper, W1_hyp)
  s1, sh = _sc_dual()(h1, x, eg, eh, zeros)
  h2, emb_hyper = _tc_b(s1, W2_gcn, sh, W_hgcn)
  s2a, s2b = _sc_single()(h2, eg, zeros)
  return _tc_cd(s2a, s2b, W3_gcn, h1, h2, emb_hyper, att, alpha, W_dec)


# speedup vs baseline: 12.8503x; 1.2233x over previous
import functools

import jax
import jax.numpy as jnp
from jax import lax
from jax.experimental import pallas as pl
from jax.experimental.pallas import tpu as pltpu
from jax.experimental.pallas import tpu_sc as plsc

N = 10000
E = 320000
D1 = 128
EMB = 64
NUM_R = 5000

NC = 2
NS = 16
CHUNK = 125
NROW = E // CHUNK
ROWS_PER_SUB = NROW // NS
ROWS_PER_WORKER = NROW // (NC * NS)
SEG = 624
SEG_LAST = N - (NS - 1) * SEG

@functools.cache
def _mesh():
  return plsc.VectorSubcoreMesh(core_axis_name="c", subcore_axis_name="s",
                                num_cores=NC, num_subcores=NS)


NBUF = 4


def _spmm_phase(table_hbm, edges_hbm, zeros_hbm, out_hbm,
                src_v, dst_v, rows, acc, semg, sems,
                sid, nrows, row_base):
  @pl.when(sid < NS - 1)
  def _():
    pltpu.sync_copy(zeros_hbm.at[pl.ds(sid * SEG, SEG)],
                    acc.at[pl.ds(sid * SEG, SEG)])

  @pl.when(sid == NS - 1)
  def _():
    pltpu.sync_copy(zeros_hbm.at[pl.ds((NS - 1) * SEG, SEG_LAST)],
                    acc.at[pl.ds((NS - 1) * SEG, SEG_LAST)])

  pltpu.sync_copy(edges_hbm.at[0, pl.ds(row_base, nrows)],
                  src_v.at[pl.ds(0, nrows)])
  pltpu.sync_copy(edges_hbm.at[1, pl.ds(row_base, nrows)],
                  dst_v.at[pl.ds(0, nrows)])
  plsc.subcore_barrier()

  for b in range(NBUF - 1):
    pltpu.async_copy(table_hbm.at[src_v.at[b]], rows[b], semg[b])

  def body(k, carry):
    for b in range(NBUF):
      j = NBUF * k + b
      pltpu.make_async_copy(table_hbm.at[src_v.at[j]], rows[b],
                            semg[b]).wait()
      pltpu.async_copy(rows[b], acc.at[dst_v.at[j]], sems[b], add=True)

      bp = (b + NBUF - 1) % NBUF

      @pl.when(j >= 1)
      def _():
        pltpu.make_async_copy(rows[bp], acc.at[dst_v.at[j - 1]],
                              sems[bp]).wait()

      bn = bp

      @pl.when(j + NBUF - 1 < nrows)
      def _():
        pltpu.async_copy(table_hbm.at[src_v.at[j + NBUF - 1]], rows[bn],
                         semg[bn])
    return carry

  lax.fori_loop(0, nrows // NBUF, body, 0)
  pltpu.make_async_copy(rows[NBUF - 1], acc.at[dst_v.at[nrows - 1]],
                        sems[NBUF - 1]).wait()
  plsc.subcore_barrier()

  @pl.when(sid < NS - 1)
  def _():
    pltpu.sync_copy(acc.at[pl.ds(sid * SEG, SEG)],
                    out_hbm.at[pl.ds(sid * SEG, SEG)])

  @pl.when(sid == NS - 1)
  def _():
    pltpu.sync_copy(acc.at[pl.ds((NS - 1) * SEG, SEG_LAST)],
                    out_hbm.at[pl.ds((NS - 1) * SEG, SEG_LAST)])


def _sc_dual_kernel(h1_hbm, x_hbm, eg_hbm, eh_hbm, zeros_hbm,
                    s1_hbm, sh_hbm, src_v, dst_v,
                    r0, r1, r2, r3, acc,
                    g0, g1, g2, g3, s0, s1, s2, s3):
  cid = lax.axis_index("c")
  sid = lax.axis_index("s")
  base = sid * ROWS_PER_SUB
  rows = (r0, r1, r2, r3)
  semg = (g0, g1, g2, g3)
  sems = (s0, s1, s2, s3)

  @pl.when(cid == 0)
  def _():
    _spmm_phase(h1_hbm, eg_hbm, zeros_hbm, s1_hbm,
                src_v, dst_v, rows, acc, semg, sems,
                sid, ROWS_PER_SUB, base)

  @pl.when(cid == 1)
  def _():
    _spmm_phase(x_hbm, eh_hbm, zeros_hbm, sh_hbm,
                src_v, dst_v, rows, acc, semg, sems,
                sid, ROWS_PER_SUB, base)


def _sc_single_kernel(h2_hbm, eg_hbm, zeros_hbm, s2a_hbm, s2b_hbm,
                      src_v, dst_v,
                      r0, r1, r2, r3, acc,
                      g0, g1, g2, g3, s0, s1, s2, s3):
  cid = lax.axis_index("c")
  sid = lax.axis_index("s")
  wid = sid * NC + cid
  base = wid * ROWS_PER_WORKER
  rows = (r0, r1, r2, r3)
  semg = (g0, g1, g2, g3)
  sems = (s0, s1, s2, s3)

  @pl.when(cid == 0)
  def _():
    _spmm_phase(h2_hbm, eg_hbm, zeros_hbm, s2a_hbm,
                src_v, dst_v, rows, acc, semg, sems,
                sid, ROWS_PER_WORKER, base)

  @pl.when(cid == 1)
  def _():
    _spmm_phase(h2_hbm, eg_hbm, zeros_hbm, s2b_hbm,
                src_v, dst_v, rows, acc, semg, sems,
                sid, ROWS_PER_WORKER, base)


@functools.cache
def _sc_dual():
  return pl.kernel(
      _sc_dual_kernel,
      out_type=(jax.ShapeDtypeStruct((N, EMB), jnp.float32),
                jax.ShapeDtypeStruct((N, EMB), jnp.float32)),
      mesh=_mesh(),
      compiler_params=pltpu.CompilerParams(use_tc_tiling_on_sc=False),
      scratch_types=(
          [pltpu.VMEM((ROWS_PER_SUB, CHUNK), jnp.int32)] * 2
          + [pltpu.VMEM((CHUNK, EMB), jnp.float32)] * 4
          + [pltpu.VMEM_SHARED((N, EMB), jnp.float32)]
          + [pltpu.SemaphoreType.DMA] * 8
      ),
  )


@functools.cache
def _sc_single():
  return pl.kernel(
      _sc_single_kernel,
      out_type=(jax.ShapeDtypeStruct((N, EMB), jnp.float32),
                jax.ShapeDtypeStruct((N, EMB), jnp.float32)),
      mesh=_mesh(),
      compiler_params=pltpu.CompilerParams(use_tc_tiling_on_sc=False),
      scratch_types=(
          [pltpu.VMEM((ROWS_PER_SUB, CHUNK), jnp.int32)] * 2
          + [pltpu.VMEM((CHUNK, EMB), jnp.float32)] * 4
          + [pltpu.VMEM_SHARED((N, EMB), jnp.float32)]
          + [pltpu.SemaphoreType.DMA] * 8
      ),
  )


def _elu(t):
  return jnp.where(t > 0, t, jnp.exp(t) - 1.0)


def _dot1x(a, b):
  return jnp.dot(a.astype(jnp.bfloat16), b.astype(jnp.bfloat16),
                 preferred_element_type=jnp.float32)


def _tc_a_body(x_ref, w1g_ref, w2_ref, xh_ref, w1h_ref, whg_ref,
               h1_ref, mg_ref, mh_ref):
  h1 = _elu(_dot1x(x_ref[...], w1g_ref[...]))
  h1_ref[...] = h1
  mg_ref[...] = _dot1x(h1, w2_ref[...])
  x = jnp.maximum(_dot1x(xh_ref[...], w1h_ref[...]), 0.0)
  mh_ref[...] = _dot1x(x, whg_ref[...])


_RB = 2000

_tc_a = pl.pallas_call(
    _tc_a_body,
    grid=(N // _RB,),
    in_specs=[
        pl.BlockSpec((_RB, D1), lambda i: (i, 0)),
        pl.BlockSpec((D1, EMB), lambda i: (0, 0)),
        pl.BlockSpec((EMB, EMB), lambda i: (0, 0)),
        pl.BlockSpec((_RB, D1), lambda i: (i, 0)),
        pl.BlockSpec((D1, EMB), lambda i: (0, 0)),
        pl.BlockSpec((EMB, EMB), lambda i: (0, 0)),
    ],
    out_specs=tuple(pl.BlockSpec((_RB, EMB), lambda i: (i, 0))
                    for _ in range(3)),
    out_shape=tuple(jax.ShapeDtypeStruct((N, EMB), jnp.float32)
                    for _ in range(3)),
)


def _tc_b_body(s1_ref, w3_ref, sh_ref, h2_ref, eh_ref, m3_ref):
  h2 = _elu(s1_ref[...])
  h2_ref[...] = h2
  m3_ref[...] = _dot1x(h2, w3_ref[...])
  eh_ref[...] = jnp.maximum(sh_ref[...], 0.0)


_tc_b = pl.pallas_call(
    _tc_b_body,
    grid=(N // _RB,),
    in_specs=[
        pl.BlockSpec((_RB, EMB), lambda i: (i, 0)),
        pl.BlockSpec((EMB, EMB), lambda i: (0, 0)),
        pl.BlockSpec((_RB, EMB), lambda i: (i, 0)),
    ],
    out_specs=tuple(pl.BlockSpec((_RB, EMB), lambda i: (i, 0))
                    for _ in range(3)),
    out_shape=tuple(jax.ShapeDtypeStruct((N, EMB), jnp.float32)
                    for _ in range(3)),
)


def _tc_cd_body(s2a_ref, s2b_ref, h1_ref, h2_ref, eh_ref,
                att_ref, alpha_ref, wdec_ref, out_ref, rw_s, dm_s):
  i = pl.program_id(0)

  @pl.when(i == 0)
  def _():
    h3 = _elu(s2a_ref[...] + s2b_ref[...])
    e0 = jnp.exp(alpha_ref[0])
    e1 = jnp.exp(alpha_ref[1])
    a0 = e0 / (e0 + e1)
    a1 = e1 / (e0 + e1)
    fused = (a0 * (att_ref[0] * h1_ref[...] + att_ref[1] * h2_ref[...]
                   + att_ref[2] * h3)
             + a1 * eh_ref[...])
    rw_s[...] = _dot1x(fused[:NUM_R], wdec_ref[...])
    dm_s[...] = fused[NUM_R:]

  t = lax.dot_general(rw_s[pl.ds(i * _BM, _BM), :].astype(jnp.bfloat16),
                      dm_s[...].astype(jnp.bfloat16),
                      dimension_numbers=(((1,), (1,)), ((), ())),
                      preferred_element_type=jnp.float32)
  out_ref[...] = 1.0 / (1.0 + jnp.exp(-t))


_BM = 200

_tc_cd = pl.pallas_call(
    _tc_cd_body,
    grid=(NUM_R // _BM,),
    in_specs=[pl.BlockSpec((N, EMB), lambda i: (0, 0)) for _ in range(2)] + [
        pl.BlockSpec((N, EMB), lambda i: (0, 0)) for _ in range(3)
    ] + [
        pl.BlockSpec(memory_space=pltpu.SMEM),
        pl.BlockSpec(memory_space=pltpu.SMEM),
        pl.BlockSpec((EMB, EMB), lambda i: (0, 0)),
    ],
    out_specs=pl.BlockSpec((_BM, NUM_R), lambda i: (i, 0)),
    out_shape=jax.ShapeDtypeStruct((NUM_R, NUM_R), jnp.float32),
    scratch_shapes=[
        pltpu.VMEM((NUM_R, EMB), jnp.float32),
        pltpu.VMEM((NUM_R, EMB), jnp.float32),
    ],
)


def kernel(inputs_data, inputs_hyper, edge_index, hyper_edge_index,
           W1_gcn, W2_gcn, W3_gcn, att, W1_hyp, W_hgcn, alpha, W_dec):
  eg = edge_index.reshape(2, NROW, CHUNK)
  eh = hyper_edge_index.reshape(2, NROW, CHUNK)
  zeros = jnp.zeros((N, EMB), jnp.float32)

  h1, mg, mh = _tc_a(inputs_data, W1_gcn, W2_gcn, inputs_hyper, W1_hyp,
                     W_hgcn)
  s1, sh = _sc_dual()(mg, mh, eg, eh, zeros)
  h2, emb_hyper, m3 = _tc_b(s1, W3_gcn, sh)
  s2a, s2b = _sc_single()(m3, eg, zeros)
  return _tc_cd(s2a, s2b, h1, h2, emb_hyper, att, alpha, W_dec)


# speedup vs baseline: 13.4055x; 1.0432x over previous
import functools

import jax
import jax.numpy as jnp
from jax import lax
from jax.experimental import pallas as pl
from jax.experimental.pallas import tpu as pltpu
from jax.experimental.pallas import tpu_sc as plsc

N = 10000
E = 320000
D1 = 128
EMB = 64
NUM_R = 5000

NC = 2
NS = 16
CHUNK = 125
NROW = E // CHUNK
ROWS_PER_SUB = NROW // NS
ROWS_PER_WORKER = NROW // (NC * NS)
SEG = 624
SEG_LAST = N - (NS - 1) * SEG

@functools.cache
def _mesh():
  return plsc.VectorSubcoreMesh(core_axis_name="c", subcore_axis_name="s",
                                num_cores=NC, num_subcores=NS)


NBUF = 5


def _spmm_phase(table_hbm, edges_hbm, zeros_hbm, out_hbm,
                src_v, dst_v, rows, acc, semg, sems,
                sid, nrows, row_base):
  @pl.when(sid < NS - 1)
  def _():
    pltpu.sync_copy(zeros_hbm.at[pl.ds(sid * SEG, SEG)],
                    acc.at[pl.ds(sid * SEG, SEG)])

  @pl.when(sid == NS - 1)
  def _():
    pltpu.sync_copy(zeros_hbm.at[pl.ds((NS - 1) * SEG, SEG_LAST)],
                    acc.at[pl.ds((NS - 1) * SEG, SEG_LAST)])

  pltpu.sync_copy(edges_hbm.at[0, pl.ds(row_base, nrows)],
                  src_v.at[pl.ds(0, nrows)])
  pltpu.sync_copy(edges_hbm.at[1, pl.ds(row_base, nrows)],
                  dst_v.at[pl.ds(0, nrows)])
  plsc.subcore_barrier()

  for b in range(NBUF - 1):
    pltpu.async_copy(table_hbm.at[src_v.at[b]], rows[b], semg[b])

  def body(k, carry):
    for b in range(NBUF):
      j = NBUF * k + b
      pltpu.make_async_copy(table_hbm.at[src_v.at[j]], rows[b],
                            semg[b]).wait()
      pltpu.async_copy(rows[b], acc.at[dst_v.at[j]], sems[b], add=True)

      bp = (b + NBUF - 1) % NBUF

      @pl.when(j >= 1)
      def _():
        pltpu.make_async_copy(rows[bp], acc.at[dst_v.at[j - 1]],
                              sems[bp]).wait()

      bn = bp

      @pl.when(j + NBUF - 1 < nrows)
      def _():
        pltpu.async_copy(table_hbm.at[src_v.at[j + NBUF - 1]], rows[bn],
                         semg[bn])
    return carry

  lax.fori_loop(0, nrows // NBUF, body, 0)
  pltpu.make_async_copy(rows[NBUF - 1], acc.at[dst_v.at[nrows - 1]],
                        sems[NBUF - 1]).wait()
  plsc.subcore_barrier()

  @pl.when(sid < NS - 1)
  def _():
    pltpu.sync_copy(acc.at[pl.ds(sid * SEG, SEG)],
                    out_hbm.at[pl.ds(sid * SEG, SEG)])

  @pl.when(sid == NS - 1)
  def _():
    pltpu.sync_copy(acc.at[pl.ds((NS - 1) * SEG, SEG_LAST)],
                    out_hbm.at[pl.ds((NS - 1) * SEG, SEG_LAST)])


def _sc_dual_kernel(h1_hbm, x_hbm, eg_hbm, eh_hbm, zeros_hbm,
                    s1_hbm, sh_hbm, src_v, dst_v, *bufs):
  cid = lax.axis_index("c")
  sid = lax.axis_index("s")
  base = sid * ROWS_PER_SUB
  rows = bufs[:NBUF]
  acc = bufs[NBUF]
  semg = bufs[NBUF + 1:NBUF + 1 + NBUF]
  sems = bufs[NBUF + 1 + NBUF:]

  @pl.when(cid == 0)
  def _():
    _spmm_phase(h1_hbm, eg_hbm, zeros_hbm, s1_hbm,
                src_v, dst_v, rows, acc, semg, sems,
                sid, ROWS_PER_SUB, base)

  @pl.when(cid == 1)
  def _():
    _spmm_phase(x_hbm, eh_hbm, zeros_hbm, sh_hbm,
                src_v, dst_v, rows, acc, semg, sems,
                sid, ROWS_PER_SUB, base)


def _sc_single_kernel(h2_hbm, eg_hbm, zeros_hbm, s2a_hbm, s2b_hbm,
                      src_v, dst_v, *bufs):
  cid = lax.axis_index("c")
  sid = lax.axis_index("s")
  wid = sid * NC + cid
  base = wid * ROWS_PER_WORKER
  rows = bufs[:NBUF]
  acc = bufs[NBUF]
  semg = bufs[NBUF + 1:NBUF + 1 + NBUF]
  sems = bufs[NBUF + 1 + NBUF:]

  @pl.when(cid == 0)
  def _():
    _spmm_phase(h2_hbm, eg_hbm, zeros_hbm, s2a_hbm,
                src_v, dst_v, rows, acc, semg, sems,
                sid, ROWS_PER_WORKER, base)

  @pl.when(cid == 1)
  def _():
    _spmm_phase(h2_hbm, eg_hbm, zeros_hbm, s2b_hbm,
                src_v, dst_v, rows, acc, semg, sems,
                sid, ROWS_PER_WORKER, base)


@functools.cache
def _sc_dual():
  return pl.kernel(
      _sc_dual_kernel,
      out_type=(jax.ShapeDtypeStruct((N, EMB), jnp.float32),
                jax.ShapeDtypeStruct((N, EMB), jnp.float32)),
      mesh=_mesh(),
      compiler_params=pltpu.CompilerParams(use_tc_tiling_on_sc=False),
      scratch_types=(
          [pltpu.VMEM((ROWS_PER_SUB, CHUNK), jnp.int32)] * 2
          + [pltpu.VMEM((CHUNK, EMB), jnp.float32)] * NBUF
          + [pltpu.VMEM_SHARED((N, EMB), jnp.float32)]
          + [pltpu.SemaphoreType.DMA] * (2 * NBUF)
      ),
  )


@functools.cache
def _sc_single():
  return pl.kernel(
      _sc_single_kernel,
      out_type=(jax.ShapeDtypeStruct((N, EMB), jnp.float32),
                jax.ShapeDtypeStruct((N, EMB), jnp.float32)),
      mesh=_mesh(),
      compiler_params=pltpu.CompilerParams(use_tc_tiling_on_sc=False),
      scratch_types=(
          [pltpu.VMEM((ROWS_PER_SUB, CHUNK), jnp.int32)] * 2
          + [pltpu.VMEM((CHUNK, EMB), jnp.float32)] * NBUF
          + [pltpu.VMEM_SHARED((N, EMB), jnp.float32)]
          + [pltpu.SemaphoreType.DMA] * (2 * NBUF)
      ),
  )


def _elu(t):
  return jnp.where(t > 0, t, jnp.exp(t) - 1.0)


def _dot1x(a, b):
  return jnp.dot(a.astype(jnp.bfloat16), b.astype(jnp.bfloat16),
                 preferred_element_type=jnp.float32)


def _tc_a_body(x_ref, w1g_ref, w2_ref, xh_ref, w1h_ref, whg_ref,
               h1_ref, mg_ref, mh_ref):
  h1 = _elu(_dot1x(x_ref[...], w1g_ref[...]))
  h1_ref[...] = h1
  mg_ref[...] = _dot1x(h1, w2_ref[...])
  x = jnp.maximum(_dot1x(xh_ref[...], w1h_ref[...]), 0.0)
  mh_ref[...] = _dot1x(x, whg_ref[...])


_RB = 2000

_tc_a = pl.pallas_call(
    _tc_a_body,
    grid=(N // _RB,),
    in_specs=[
        pl.BlockSpec((_RB, D1), lambda i: (i, 0)),
        pl.BlockSpec((D1, EMB), lambda i: (0, 0)),
        pl.BlockSpec((EMB, EMB), lambda i: (0, 0)),
        pl.BlockSpec((_RB, D1), lambda i: (i, 0)),
        pl.BlockSpec((D1, EMB), lambda i: (0, 0)),
        pl.BlockSpec((EMB, EMB), lambda i: (0, 0)),
    ],
    out_specs=tuple(pl.BlockSpec((_RB, EMB), lambda i: (i, 0))
                    for _ in range(3)),
    out_shape=tuple(jax.ShapeDtypeStruct((N, EMB), jnp.float32)
                    for _ in range(3)),
)


def _tc_b_body(s1_ref, w3_ref, sh_ref, h2_ref, eh_ref, m3_ref):
  h2 = _elu(s1_ref[...])
  h2_ref[...] = h2
  m3_ref[...] = _dot1x(h2, w3_ref[...])
  eh_ref[...] = jnp.maximum(sh_ref[...], 0.0)


_tc_b = pl.pallas_call(
    _tc_b_body,
    grid=(N // _RB,),
    in_specs=[
        pl.BlockSpec((_RB, EMB), lambda i: (i, 0)),
        pl.BlockSpec((EMB, EMB), lambda i: (0, 0)),
        pl.BlockSpec((_RB, EMB), lambda i: (i, 0)),
    ],
    out_specs=tuple(pl.BlockSpec((_RB, EMB), lambda i: (i, 0))
                    for _ in range(3)),
    out_shape=tuple(jax.ShapeDtypeStruct((N, EMB), jnp.float32)
                    for _ in range(3)),
)


def _tc_cd_body(s2a_ref, s2b_ref, h1_ref, h2_ref, eh_ref,
                att_ref, alpha_ref, wdec_ref, out_ref, rw_s, dm_s):
  i = pl.program_id(0)

  @pl.when(i == 0)
  def _():
    h3 = _elu(s2a_ref[...] + s2b_ref[...])
    e0 = jnp.exp(alpha_ref[0])
    e1 = jnp.exp(alpha_ref[1])
    a0 = e0 / (e0 + e1)
    a1 = e1 / (e0 + e1)
    fused = (a0 * (att_ref[0] * h1_ref[...] + att_ref[1] * h2_ref[...]
                   + att_ref[2] * h3)
             + a1 * eh_ref[...])
    rw_s[...] = _dot1x(fused[:NUM_R], wdec_ref[...])
    dm_s[...] = fused[NUM_R:]

  t = lax.dot_general(rw_s[pl.ds(i * _BM, _BM), :].astype(jnp.bfloat16),
                      dm_s[...].astype(jnp.bfloat16),
                      dimension_numbers=(((1,), (1,)), ((), ())),
                      preferred_element_type=jnp.float32)
  out_ref[...] = 1.0 / (1.0 + jnp.exp(-t))


_BM = 200

_tc_cd = pl.pallas_call(
    _tc_cd_body,
    grid=(NUM_R // _BM,),
    in_specs=[pl.BlockSpec((N, EMB), lambda i: (0, 0)) for _ in range(2)] + [
        pl.BlockSpec((N, EMB), lambda i: (0, 0)) for _ in range(3)
    ] + [
        pl.BlockSpec(memory_space=pltpu.SMEM),
        pl.BlockSpec(memory_space=pltpu.SMEM),
        pl.BlockSpec((EMB, EMB), lambda i: (0, 0)),
    ],
    out_specs=pl.BlockSpec((_BM, NUM_R), lambda i: (i, 0)),
    out_shape=jax.ShapeDtypeStruct((NUM_R, NUM_R), jnp.float32),
    scratch_shapes=[
        pltpu.VMEM((NUM_R, EMB), jnp.float32),
        pltpu.VMEM((NUM_R, EMB), jnp.float32),
    ],
)


def kernel(inputs_data, inputs_hyper, edge_index, hyper_edge_index,
           W1_gcn, W2_gcn, W3_gcn, att, W1_hyp, W_hgcn, alpha, W_dec):
  eg = edge_index.reshape(2, NROW, CHUNK)
  eh = hyper_edge_index.reshape(2, NROW, CHUNK)
  zeros = jnp.zeros((N, EMB), jnp.float32)

  h1, mg, mh = _tc_a(inputs_data, W1_gcn, W2_gcn, inputs_hyper, W1_hyp,
                     W_hgcn)
  s1, sh = _sc_dual()(mg, mh, eg, eh, zeros)
  h2, emb_hyper, m3 = _tc_b(s1, W3_gcn, sh)
  s2a, s2b = _sc_single()(m3, eg, zeros)
  return _tc_cd(s2a, s2b, h1, h2, emb_hyper, att, alpha, W_dec)
